# baseline (speedup 1.0000x reference)
import functools

import jax
import jax.numpy as jnp
from jax.experimental import pallas as pl
from jax.experimental.pallas import tpu as pltpu

T = 1024
I = 100
U = 100000
P = 64

BU = 2048
NU = (U + BU - 1) // BU

BT2 = 256


def _matmul_kernel(oh_ref, coef_ref, cu_ref):
    j = pl.program_id(0)

    @pl.when(j == 0)
    def _init():
        cu_ref[...] = jnp.zeros_like(cu_ref)

    oh = oh_ref[...]
    cf = coef_ref[...]
    limit = U - j * BU
    col = jax.lax.broadcasted_iota(jnp.int32, oh.shape, 1)
    oh = jnp.where(col < limit, oh, 0.0)
    row = jax.lax.broadcasted_iota(jnp.int32, cf.shape, 0)
    cf = jnp.where(row < limit, cf, 0.0)
    cu_ref[...] += jnp.dot(oh, cf, preferred_element_type=jnp.float32)


def _epilogue_kernel(x_ref, cu_ref, out_ref):
    out_ref[...] = jnp.sum(x_ref[...] * cu_ref[...][:, None, :], axis=-1)


@jax.jit
def kernel(x, user_onehot, coef):
    oh = user_onehot.reshape(T, U)

    cu = pl.pallas_call(
        _matmul_kernel,
        grid=(NU,),
        in_specs=[
            pl.BlockSpec((T, BU), lambda j: (0, j)),
            pl.BlockSpec((BU, P), lambda j: (j, 0)),
        ],
        out_specs=pl.BlockSpec((T, P), lambda j: (0, 0)),
        out_shape=jax.ShapeDtypeStruct((T, P), jnp.float32),
        compiler_params=pltpu.CompilerParams(
            dimension_semantics=("arbitrary",),
        ),
    )(oh, coef)

    out = pl.pallas_call(
        _epilogue_kernel,
        grid=(T // BT2,),
        in_specs=[
            pl.BlockSpec((BT2, I, P), lambda i: (i, 0, 0)),
            pl.BlockSpec((BT2, P), lambda i: (i, 0)),
        ],
        out_specs=pl.BlockSpec((BT2, I), lambda i: (i, 0)),
        out_shape=jax.ShapeDtypeStruct((T, I), jnp.float32),
        compiler_params=pltpu.CompilerParams(
            dimension_semantics=("parallel",),
        ),
    )(x, cu)

    return out
